# baseline (speedup 1.0000x reference)
import functools

import jax
import jax.numpy as jnp
from jax import lax
from jax.experimental import pallas as pl
from jax.experimental.pallas import tpu as pltpu
from jax.experimental.pallas import tpu_sc as plsc

_B = 16384
_F = 32
_NW = 32
_BPW = _B // _NW
_NCHUNK = 4
_CHUNK = _BPW // _NCHUNK
_NGROUP = _BPW // 16


def _sc_body(user_ref, item_ref, uemb_ref, iemb_ref, w_ref, b_ref, out_ref,
             uidx, iidx, urows, irows, wv, bv, outv, sem):
    nc = 2
    wid = lax.axis_index("s") * nc + lax.axis_index("c")

    pltpu.sync_copy(user_ref.at[wid], uidx)
    pltpu.sync_copy(item_ref.at[wid], iidx)
    pltpu.sync_copy(w_ref, wv)
    pltpu.sync_copy(b_ref, bv)

    copies = []
    for j in range(_NCHUNK):
        copies.append(pltpu.async_copy(
            uemb_ref.at[uidx.at[j]], urows.at[pl.ds(j * _CHUNK, _CHUNK)], sem))
        copies.append(pltpu.async_copy(
            iemb_ref.at[iidx.at[j]], irows.at[pl.ds(j * _CHUNK, _CHUNK)], sem))
    for cp in copies:
        cp.wait()

    iot = lax.iota(jnp.int32, 16)
    w_vec = wv[...]
    b_vec = bv[...]

    def group(g, carry):
        rows = g * 16 + iot
        acc = jnp.zeros((16,), jnp.float32)
        for f in range(_F):
            fcol = jnp.full((16,), f, jnp.int32)
            u = plsc.load_gather(urows, [rows, fcol])
            i = plsc.load_gather(irows, [rows, fcol])
            d = u - i
            acc = acc + d * d
        half = acc * 0.5
        bits = plsc.bitcast(acc, jnp.int32)
        bits = jnp.int32(0x5F3759DF) - (bits >> 1)
        y = plsc.bitcast(bits, jnp.float32)
        for _ in range(3):
            y = y * (1.5 - half * y * y)
        p2 = acc * y
        outv[pl.ds(g * 16, 16)] = p2 * w_vec + b_vec
        return carry

    lax.fori_loop(0, _NGROUP, group, 0)
    pltpu.sync_copy(outv, out_ref.at[pl.ds(wid * _BPW, _BPW)])


@functools.partial(
    pl.kernel,
    mesh=plsc.VectorSubcoreMesh(core_axis_name="c", subcore_axis_name="s"),
    out_type=jax.ShapeDtypeStruct((_B,), jnp.float32),
    compiler_params=pltpu.CompilerParams(
        needs_layout_passes=False, use_tc_tiling_on_sc=False),
    scratch_types=[
        pltpu.VMEM((_NCHUNK, _CHUNK), jnp.int32),
        pltpu.VMEM((_NCHUNK, _CHUNK), jnp.int32),
        pltpu.VMEM((_BPW, _F), jnp.float32),
        pltpu.VMEM((_BPW, _F), jnp.float32),
        pltpu.VMEM((16,), jnp.float32),
        pltpu.VMEM((16,), jnp.float32),
        pltpu.VMEM((_BPW,), jnp.float32),
        pltpu.SemaphoreType.DMA,
    ],
)
def _sc_rating(user_ref, item_ref, uemb_ref, iemb_ref, w_ref, b_ref, out_ref,
               uidx, iidx, urows, irows, wv, bv, outv, sem):
    _sc_body(user_ref, item_ref, uemb_ref, iemb_ref, w_ref, b_ref, out_ref,
             uidx, iidx, urows, irows, wv, bv, outv, sem)


def kernel(user, item, user_emb, item_emb, lin_w, lin_b):
    user_r = user.astype(jnp.int32).reshape(_NW, _NCHUNK, _CHUNK)
    item_r = item.astype(jnp.int32).reshape(_NW, _NCHUNK, _CHUNK)
    w16 = jnp.full((16,), lin_w.reshape(()), jnp.float32)
    b16 = jnp.full((16,), lin_b.reshape(()), jnp.float32)
    return _sc_rating(user_r, item_r, user_emb, item_emb, w16, b16)
